# baseline (speedup 1.0000x reference)
import jax
import jax.numpy as jnp
from jax import lax
from jax.experimental import pallas as pl
from jax.experimental.pallas import tpu as pltpu
from jax.experimental.pallas import tpu_sc as plsc

N = 1600000
D = 15
S = 4096
OUT = 5
YD = 8

NC = 2
NS = 16
NW = NC * NS
LANES = 16
ROWS_PER_TILE = N // NW
CHUNK = 10000
NCHUNKS = ROWS_PER_TILE // CHUNK

PROJ_PACK = 8
PROJ_BLK = 1000


def _tc_proj_body(x_ref, wbd_ref, bt_ref, o_ref):
    y = jnp.dot(x_ref[...], wbd_ref[...], preferred_element_type=jnp.float32)
    col = lax.broadcasted_iota(jnp.int32, y.shape, 1)
    bt = jnp.repeat(bt_ref[...].astype(jnp.float32), YD, axis=1)
    y = jnp.where(col % YD == OUT, 1.0, y)
    o_ref[...] = jnp.where(col % YD == OUT + 1, bt, y)


_tc_project = pl.pallas_call(
    _tc_proj_body,
    grid=(N // PROJ_PACK // PROJ_BLK,),
    in_specs=[
        pl.BlockSpec((PROJ_BLK, PROJ_PACK * D), lambda i: (i, 0)),
        pl.BlockSpec((PROJ_PACK * D, PROJ_PACK * YD), lambda i: (0, 0)),
        pl.BlockSpec((PROJ_BLK, PROJ_PACK), lambda i: (i, 0)),
    ],
    out_specs=pl.BlockSpec((PROJ_BLK, PROJ_PACK * YD), lambda i: (i, 0)),
    out_shape=jax.ShapeDtypeStruct((N // PROJ_PACK, PROJ_PACK * YD),
                                   jnp.float32),
)


def _sc_body(y_hbm, out_hbm, ybuf, acc):
    wid = lax.axis_index("s") * NC + lax.axis_index("c")
    base_row = wid * ROWS_PER_TILE
    iota = lax.iota(jnp.int32, LANES)
    iota7 = jnp.bitwise_and(iota, 7)
    iota8 = iota * YD
    low_half = iota < YD
    zeros = jnp.zeros((LANES,), jnp.float32)

    @plsc.parallel_loop(0, S * YD, step=LANES, unroll=4)
    def _zero(j):
        acc[pl.ds(j, LANES)] = zeros

    def _flush(acc_reg, cur_seg):
        seg = cur_seg.astype(jnp.int32)
        idx = jnp.full((LANES,), seg * YD, jnp.int32) + iota7
        plsc.addupdate_scatter(acc, [idx], acc_reg)

    def group_body(g, carry):
        acc_reg, cur_seg = carry
        g0 = g * LANES
        bvec = plsc.load_gather(
            ybuf, [jnp.full((LANES,), g0 * YD + OUT + 1, jnp.int32) + iota8])
        in_run = bvec == jnp.full((LANES,), cur_seg)
        all_same = plsc.all_reduce_population_count(in_run)[0] == LANES

        def fast(carry):
            acc_reg, cur_seg = carry
            vs = [ybuf[pl.ds(g0 * YD + LANES * k, LANES)] for k in range(8)]
            while len(vs) > 1:
                vs = [a + b for a, b in zip(vs[::2], vs[1::2])]
            return acc_reg + vs[0], cur_seg

        def slow(carry):
            acc_reg, cur_seg = carry
            _flush(acc_reg, cur_seg)
            for k in range(8):
                v = ybuf[pl.ds(g0 * YD + LANES * k, LANES)]
                sa = bvec[2 * k]
                sb = bvec[2 * k + 1]
                segv = jnp.where(low_half, jnp.full((LANES,), sa),
                                 jnp.full((LANES,), sb)).astype(jnp.int32)
                plsc.addupdate_scatter(acc, [segv * YD + iota7], v)
            return zeros, bvec[LANES - 1]

        return lax.cond(all_same, fast, slow, (acc_reg, cur_seg))

    for c in range(NCHUNKS):
        r0 = base_row + c * CHUNK
        pltpu.sync_copy(y_hbm.at[pl.ds(r0 * YD, CHUNK * YD)], ybuf)
        cur_seg0 = plsc.load_gather(
            ybuf, [jnp.full((LANES,), OUT + 1, jnp.int32) + iota8])[0]
        acc_reg, cur_seg = lax.fori_loop(
            0, CHUNK // LANES, group_body, (zeros, cur_seg0))
        _flush(acc_reg, cur_seg)

    pltpu.sync_copy(acc, out_hbm.at[wid])


_sc_segment_sum = pl.kernel(
    _sc_body,
    out_type=jax.ShapeDtypeStruct((NW, S * YD), jnp.float32),
    mesh=plsc.VectorSubcoreMesh(core_axis_name="c", subcore_axis_name="s"),
    compiler_params=pltpu.CompilerParams(needs_layout_passes=False),
    scratch_types=[
        pltpu.VMEM((CHUNK * YD,), jnp.float32),
        pltpu.VMEM((S * YD,), jnp.float32),
    ],
)


def _tc_tail_body(p_ref, b_ref, o_ref, acc_ref):
    i = pl.program_id(0)

    @pl.when(i == 0)
    def _init():
        acc_ref[...] = p_ref[0]

    @pl.when(i > 0)
    def _accum():
        acc_ref[...] += p_ref[0]

    @pl.when(i == NW - 1)
    def _finish():
        s = acc_ref[...]
        counts = s[:, OUT]
        mean = s[:, :OUT] / jnp.maximum(counts, 1.0)[:, None]
        o_ref[...] = jnp.where(counts[:, None] > 0,
                               mean + b_ref[...][None, :], 0.0)


_tc_tail = pl.pallas_call(
    _tc_tail_body,
    grid=(NW,),
    in_specs=[
        pl.BlockSpec((1, S, YD), lambda i: (i, 0, 0)),
        pl.BlockSpec((OUT,), lambda i: (0,)),
    ],
    out_specs=pl.BlockSpec((S, OUT), lambda i: (0, 0)),
    scratch_shapes=[pltpu.VMEM((S, YD), jnp.float32)],
    out_shape=jax.ShapeDtypeStruct((S, OUT), jnp.float32),
)


def kernel(x, batch, W, b):
    wt8 = jnp.pad(W.T, ((0, 0), (0, YD - OUT)))
    wbd = jnp.kron(jnp.eye(PROJ_PACK, dtype=x.dtype), wt8)
    batch2 = batch.astype(jnp.int32).reshape(N // PROJ_PACK, PROJ_PACK)
    y = _tc_project(x.reshape(N // PROJ_PACK, PROJ_PACK * D), wbd, batch2)
    partials = _sc_segment_sum(y.reshape(N * YD))
    return _tc_tail(partials.reshape(NW, S, YD), b)


# speedup vs baseline: 1.1183x; 1.1183x over previous
import jax
import jax.numpy as jnp
from jax import lax
from jax.experimental import pallas as pl
from jax.experimental.pallas import tpu as pltpu
from jax.experimental.pallas import tpu_sc as plsc

N = 1600000
D = 15
S = 4096
OUT = 5
YD = 8

NC = 2
NS = 16
NW = NC * NS
LANES = 16
ROWS_PER_TILE = N // NW
CHUNK = 10000
NCHUNKS = ROWS_PER_TILE // CHUNK

PROJ_PACK = 8
PROJ_BLK = 1000


def _tc_proj_body(x_ref, wbd_ref, bt_ref, o_ref):
    y = jnp.dot(x_ref[...], wbd_ref[...], preferred_element_type=jnp.float32)
    col = lax.broadcasted_iota(jnp.int32, y.shape, 1)
    bt = jnp.repeat(bt_ref[...].astype(jnp.float32), YD, axis=1)
    y = jnp.where(col % YD == OUT, 1.0, y)
    o_ref[...] = jnp.where(col % YD == OUT + 1, bt, y)


_tc_project = pl.pallas_call(
    _tc_proj_body,
    grid=(N // PROJ_PACK // PROJ_BLK,),
    in_specs=[
        pl.BlockSpec((PROJ_BLK, PROJ_PACK * D), lambda i: (i, 0)),
        pl.BlockSpec((PROJ_PACK * D, PROJ_PACK * YD), lambda i: (0, 0)),
        pl.BlockSpec((PROJ_BLK, PROJ_PACK), lambda i: (i, 0)),
    ],
    out_specs=pl.BlockSpec((PROJ_BLK, PROJ_PACK * YD), lambda i: (i, 0)),
    out_shape=jax.ShapeDtypeStruct((N // PROJ_PACK, PROJ_PACK * YD),
                                   jnp.float32),
)


def _sc_body(y_hbm, out_hbm, ybuf, acc):
    wid = lax.axis_index("s") * NC + lax.axis_index("c")
    base_row = wid * ROWS_PER_TILE
    iota = lax.iota(jnp.int32, LANES)
    iota7 = jnp.bitwise_and(iota, 7)
    iota8 = iota * YD
    low_half = iota < YD
    zeros = jnp.zeros((LANES,), jnp.float32)

    @plsc.parallel_loop(0, S * YD, step=LANES, unroll=4)
    def _zero(j):
        acc[pl.ds(j, LANES)] = zeros

    def _flush(acc_reg, cur_seg):
        seg = cur_seg.astype(jnp.int32)
        idx = jnp.full((LANES,), seg * YD, jnp.int32) + iota7
        plsc.addupdate_scatter(acc, [idx], acc_reg)

    def group_body(g, carry):
        acc_reg, cur_seg = carry
        g0 = g * LANES
        bvec = plsc.load_gather(
            ybuf, [jnp.full((LANES,), g0 * YD + OUT + 1, jnp.int32) + iota8])
        in_run = bvec == jnp.full((LANES,), cur_seg)
        all_same = plsc.all_reduce_population_count(in_run)[0] == LANES

        def fast(carry):
            acc_reg, cur_seg = carry
            vs = [ybuf[pl.ds(g0 * YD + LANES * k, LANES)] for k in range(8)]
            while len(vs) > 1:
                vs = [a + b for a, b in zip(vs[::2], vs[1::2])]
            return acc_reg + vs[0], cur_seg

        def slow(carry):
            acc_reg, cur_seg = carry
            _flush(acc_reg, cur_seg)
            for k in range(8):
                v = ybuf[pl.ds(g0 * YD + LANES * k, LANES)]
                sa = bvec[2 * k]
                sb = bvec[2 * k + 1]
                segv = jnp.where(low_half, jnp.full((LANES,), sa),
                                 jnp.full((LANES,), sb)).astype(jnp.int32)
                plsc.addupdate_scatter(acc, [segv * YD + iota7], v)
            return zeros, bvec[LANES - 1]

        return lax.cond(all_same, fast, slow, (acc_reg, cur_seg))

    pltpu.sync_copy(acc, out_hbm.at[wid])


_sc_segment_sum = pl.kernel(
    _sc_body,
    out_type=jax.ShapeDtypeStruct((NW, S * YD), jnp.float32),
    mesh=plsc.VectorSubcoreMesh(core_axis_name="c", subcore_axis_name="s"),
    compiler_params=pltpu.CompilerParams(needs_layout_passes=False),
    scratch_types=[
        pltpu.VMEM((CHUNK * YD,), jnp.float32),
        pltpu.VMEM((S * YD,), jnp.float32),
    ],
)


def _tc_tail_body(p_ref, b_ref, o_ref, acc_ref):
    i = pl.program_id(0)

    @pl.when(i == 0)
    def _init():
        acc_ref[...] = p_ref[0]

    @pl.when(i > 0)
    def _accum():
        acc_ref[...] += p_ref[0]

    @pl.when(i == NW - 1)
    def _finish():
        s = acc_ref[...]
        counts = s[:, OUT]
        mean = s[:, :OUT] / jnp.maximum(counts, 1.0)[:, None]
        o_ref[...] = jnp.where(counts[:, None] > 0,
                               mean + b_ref[...][None, :], 0.0)


_tc_tail = pl.pallas_call(
    _tc_tail_body,
    grid=(NW,),
    in_specs=[
        pl.BlockSpec((1, S, YD), lambda i: (i, 0, 0)),
        pl.BlockSpec((OUT,), lambda i: (0,)),
    ],
    out_specs=pl.BlockSpec((S, OUT), lambda i: (0, 0)),
    scratch_shapes=[pltpu.VMEM((S, YD), jnp.float32)],
    out_shape=jax.ShapeDtypeStruct((S, OUT), jnp.float32),
)


def kernel(x, batch, W, b):
    wt8 = jnp.pad(W.T, ((0, 0), (0, YD - OUT)))
    wbd = jnp.kron(jnp.eye(PROJ_PACK, dtype=x.dtype), wt8)
    batch2 = batch.astype(jnp.int32).reshape(N // PROJ_PACK, PROJ_PACK)
    y = _tc_project(x.reshape(N // PROJ_PACK, PROJ_PACK * D), wbd, batch2)
    partials = _sc_segment_sum(y.reshape(N * YD))
    return _tc_tail(partials.reshape(NW, S, YD), b)


# speedup vs baseline: 1.3331x; 1.1921x over previous
import jax
import jax.numpy as jnp
from jax import lax
from jax.experimental import pallas as pl
from jax.experimental.pallas import tpu as pltpu
from jax.experimental.pallas import tpu_sc as plsc

N = 1600000
D = 15
S = 4096
OUT = 5

NC = 2
NS = 16
NW = NC * NS
LANES = 16
ROWS_PER_TILE = N // NW
CHUNK = 2000
NCHUNKS = ROWS_PER_TILE // CHUNK


def _sc_body(x_hbm, batch_hbm, out_hbm, xbuf, bbuf, acc):
    wid = lax.axis_index("s") * NC + lax.axis_index("c")
    base_row = wid * ROWS_PER_TILE
    iota = lax.iota(jnp.int32, LANES)
    lane_is_feat = iota < D
    ones = jnp.ones((LANES,), jnp.float32)
    zeros = jnp.zeros((LANES,), jnp.float32)

    @plsc.parallel_loop(0, LANES * S, step=LANES, unroll=4)
    def _zero(j):
        acc[pl.ds(j, LANES)] = zeros

    def chunk_body(c, _):
        r0 = base_row + c * CHUNK
        pltpu.sync_copy(x_hbm.at[pl.ds(r0 * D, CHUNK * D)], xbuf.at[pl.ds(0, CHUNK * D)])
        pltpu.sync_copy(batch_hbm.at[pl.ds(r0, CHUNK)], bbuf)

        @plsc.parallel_loop(0, CHUNK // LANES, unroll=2)
        def _groups(g):
            g0 = g * LANES
            bvec = bbuf[pl.ds(g0, LANES)] * LANES
            for k in range(LANES):
                row = xbuf[pl.ds((g0 + k) * D, LANES)]
                vals = jnp.where(lane_is_feat, row, ones)
                plsc.addupdate_scatter(
                    acc, [jnp.full((LANES,), bvec[k], jnp.int32) + iota], vals)
        return 0
    lax.fori_loop(0, NCHUNKS, chunk_body, 0)

    pltpu.sync_copy(acc, out_hbm.at[wid])


_sc_segment_sum = pl.kernel(
    _sc_body,
    out_type=jax.ShapeDtypeStruct((NW, S * LANES), jnp.float32),
    mesh=plsc.VectorSubcoreMesh(core_axis_name="c", subcore_axis_name="s"),
    compiler_params=pltpu.CompilerParams(needs_layout_passes=False),
    scratch_types=[
        pltpu.VMEM((CHUNK * D + LANES,), jnp.float32),
        pltpu.VMEM((CHUNK,), jnp.int32),
        pltpu.VMEM((LANES * S,), jnp.float32),
    ],
)


def _tc_tail_body(p_ref, w_ref, b_ref, o_ref, acc_ref):
    i = pl.program_id(0)

    @pl.when(i == 0)
    def _init():
        acc_ref[...] = p_ref[0]

    @pl.when(i > 0)
    def _accum():
        acc_ref[...] += p_ref[0]

    @pl.when(i == NW - 1)
    def _finish():
        s = acc_ref[...]
        counts = s[:, D]
        mean = s[:, :D] / jnp.maximum(counts, 1.0)[:, None]
        out = lax.dot_general(mean, w_ref[...], (((1,), (1,)), ((), ())),
                              preferred_element_type=jnp.float32)
        o_ref[...] = jnp.where(counts[:, None] > 0, out + b_ref[...][None, :], 0.0)


_tc_tail = pl.pallas_call(
    _tc_tail_body,
    grid=(NW,),
    in_specs=[
        pl.BlockSpec((1, S, LANES), lambda i: (i, 0, 0)),
        pl.BlockSpec((OUT, D), lambda i: (0, 0)),
        pl.BlockSpec((OUT,), lambda i: (0,)),
    ],
    out_specs=pl.BlockSpec((S, OUT), lambda i: (0, 0)),
    scratch_shapes=[pltpu.VMEM((S, LANES), jnp.float32)],
    out_shape=jax.ShapeDtypeStruct((S, OUT), jnp.float32),
)


def kernel(x, batch, W, b):
    partials = _sc_segment_sum(x.reshape(N * D), batch.astype(jnp.int32))
    return _tc_tail(partials.reshape(NW, S, LANES), W, b)
